# speedup vs baseline: 3.4889x; 3.4889x over previous
import functools

import jax
import jax.numpy as jnp
from jax import lax
from jax.experimental import pallas as pl
from jax.experimental.pallas import tpu as pltpu
from jax.experimental.pallas import tpu_sc as plsc

B = 16384
D = 256
NNE = 10
L = 16

_info = plsc.get_sparse_core_info()
NC = _info.num_cores
NS = _info.num_subcores
NW = NC * NS
BPW = B // NW

CH = 8
GI = CH * NNE
NIT = BPW // CH
SG = 128
NSG = BPW // SG

_mesh = plsc.VectorSubcoreMesh(core_axis_name="c", subcore_axis_name="s")


@functools.partial(
    pl.kernel,
    mesh=_mesh,
    out_type=(
        jax.ShapeDtypeStruct((B, D), jnp.float32),
        jax.ShapeDtypeStruct((B, D), jnp.float32),
    ),
    scratch_types=[
        pltpu.VMEM((NSG, SG), jnp.int32),
        pltpu.VMEM((NIT, GI), jnp.int32),
        pltpu.VMEM((SG, D), jnp.float32),
        pltpu.VMEM((GI, D), jnp.float32),
        pltpu.VMEM((CH, D), jnp.float32),
        pltpu.SemaphoreType.DMA,
    ],
)
def _sc_gather_mean(nodes_hbm, neigh_hbm, table_hbm, self_out, mean_out,
                    sidx_v, nidx_v, srows_v, nrows_v, mean_v, sem):
    wid = lax.axis_index("s") * NC + lax.axis_index("c")
    base = wid * BPW

    pltpu.sync_copy(nodes_hbm.at[wid], sidx_v)
    pltpu.sync_copy(neigh_hbm.at[wid], nidx_v)

    for g in range(NSG):
        pltpu.async_copy(table_hbm.at[sidx_v.at[g]], srows_v, sem).wait()
        pltpu.sync_copy(srows_v, self_out.at[pl.ds(base + g * SG, SG)])

    def chunk_body(it, _):
        pltpu.async_copy(table_hbm.at[nidx_v.at[it]], nrows_v, sem).wait()

        def node_body(n, _):
            def col_body(d, _):
                col = pl.ds(d * L, L)
                acc = nrows_v[n * NNE, col]
                for j in range(1, NNE):
                    acc = acc + nrows_v[n * NNE + j, col]
                mean_v[n, col] = acc * (1.0 / NNE)
                return 0

            lax.fori_loop(0, D // L, col_body, 0)
            return 0

        lax.fori_loop(0, CH, node_body, 0)
        pltpu.sync_copy(mean_v, mean_out.at[pl.ds(base + it * CH, CH)])
        return 0

    lax.fori_loop(0, NIT, chunk_body, 0)


def _mm_body(s_ref, m_ref, w1_ref, w2_ref, o_ref):
    acc = jnp.dot(s_ref[...], w1_ref[...], preferred_element_type=jnp.float32)
    acc += jnp.dot(m_ref[...], w2_ref[...], preferred_element_type=jnp.float32)
    o_ref[...] = jnp.maximum(acc, 0.0)


_BM = 1024


@jax.jit
def kernel(feat_table, nodes, neigh_idx, weight):
    nodes_r = nodes.astype(jnp.int32).reshape(NW, NSG, SG)
    neigh_r = neigh_idx.astype(jnp.int32).reshape(NW, NIT, GI)

    self_f, mean_f = _sc_gather_mean(nodes_r, neigh_r, feat_table)

    out = pl.pallas_call(
        _mm_body,
        grid=(B // _BM,),
        in_specs=[
            pl.BlockSpec((_BM, D), lambda i: (i, 0)),
            pl.BlockSpec((_BM, D), lambda i: (i, 0)),
            pl.BlockSpec((D, D), lambda i: (0, 0)),
            pl.BlockSpec((D, D), lambda i: (0, 0)),
        ],
        out_specs=pl.BlockSpec((_BM, D), lambda i: (i, 0)),
        out_shape=jax.ShapeDtypeStruct((B, D), jnp.float32),
    )(self_f, mean_f, weight[:D], weight[D:])
    return out


# speedup vs baseline: 5.1525x; 1.4768x over previous
import functools

import jax
import jax.numpy as jnp
from jax import lax
from jax.experimental import pallas as pl
from jax.experimental.pallas import tpu as pltpu
from jax.experimental.pallas import tpu_sc as plsc

B = 16384
D = 256
NNE = 10
L = 16

_info = plsc.get_sparse_core_info()
NC = _info.num_cores
NS = _info.num_subcores
NW = NC * NS
BPW = B // NW

CH = 8
GI = CH * NNE
NIT = BPW // CH
SG = 64
NSG = BPW // SG
MG = 16
NGRP = NIT // MG

_mesh = plsc.VectorSubcoreMesh(core_axis_name="c", subcore_axis_name="s")


@functools.partial(
    pl.kernel,
    mesh=_mesh,
    out_type=(
        jax.ShapeDtypeStruct((B, D), jnp.float32),
        jax.ShapeDtypeStruct((B, D), jnp.float32),
    ),
    scratch_types=[
        pltpu.VMEM((NSG, SG), jnp.int32),
        pltpu.VMEM((NIT, GI), jnp.int32),
        pltpu.VMEM((SG, D), jnp.float32),
        pltpu.VMEM((SG, D), jnp.float32),
        pltpu.VMEM((GI, D), jnp.float32),
        pltpu.VMEM((GI, D), jnp.float32),
        pltpu.VMEM((MG * CH, D), jnp.float32),
        pltpu.SemaphoreType.DMA,
        pltpu.SemaphoreType.DMA,
    ],
)
def _sc_gather_mean(nodes_hbm, neigh_hbm, table_hbm, self_out, mean_out,
                    sidx_v, nidx_v, srows0, srows1, nbuf0, nbuf1, mean_v,
                    sem0, sem1):
    wid = lax.axis_index("s") * NC + lax.axis_index("c")
    base = wid * BPW

    pltpu.sync_copy(nodes_hbm.at[wid], sidx_v)
    pltpu.sync_copy(neigh_hbm.at[wid], nidx_v)

    sbufs = (srows0, srows1)
    sems = (sem0, sem1)

    pltpu.make_async_copy(table_hbm.at[sidx_v.at[0]], srows0, sem0).start()
    for g in range(NSG):
        b = g % 2
        if g + 1 < NSG:
            nb = (g + 1) % 2
            pltpu.make_async_copy(
                table_hbm.at[sidx_v.at[g + 1]], sbufs[nb], sems[nb]).start()
        pltpu.make_async_copy(
            table_hbm.at[sidx_v.at[g]], sbufs[b], sems[b]).wait()
        pltpu.sync_copy(sbufs[b], self_out.at[pl.ds(base + g * SG, SG)])

    nbufs = (nbuf0, nbuf1)
    pltpu.make_async_copy(table_hbm.at[nidx_v.at[0]], nbuf0, sem0).start()
    pltpu.make_async_copy(table_hbm.at[nidx_v.at[1]], nbuf1, sem1).start()

    def group_body(grp, _):
        def pair_body(p, _):
            for b in range(2):
                it = grp * MG + p * 2 + b
                pltpu.make_async_copy(
                    table_hbm.at[nidx_v.at[it]], nbufs[b], sems[b]).wait()

                def node_body(n, _):
                    row = (p * 2 + b) * CH + n
                    for d in range(D // L):
                        col = pl.ds(d * L, L)
                        acc = nbufs[b][n * NNE, col]
                        for j in range(1, NNE):
                            acc = acc + nbufs[b][n * NNE + j, col]
                        mean_v[row, col] = acc * (1.0 / NNE)
                    return 0

                lax.fori_loop(0, CH, node_body, 0)

                @pl.when(it + 2 < NIT)
                def _():
                    pltpu.make_async_copy(
                        table_hbm.at[nidx_v.at[it + 2]], nbufs[b],
                        sems[b]).start()
            return 0

        lax.fori_loop(0, MG // 2, pair_body, 0)
        pltpu.sync_copy(mean_v, mean_out.at[pl.ds(base + grp * (MG * CH),
                                                  MG * CH)])
        return 0

    lax.fori_loop(0, NGRP, group_body, 0)


def _mm_body(s_ref, m_ref, w1_ref, w2_ref, o_ref):
    acc = jnp.dot(s_ref[...], w1_ref[...], preferred_element_type=jnp.float32)
    acc += jnp.dot(m_ref[...], w2_ref[...], preferred_element_type=jnp.float32)
    o_ref[...] = jnp.maximum(acc, 0.0)


_BM = 1024


@jax.jit
def kernel(feat_table, nodes, neigh_idx, weight):
    nodes_r = nodes.astype(jnp.int32).reshape(NW, NSG, SG)
    neigh_r = neigh_idx.astype(jnp.int32).reshape(NW, NIT, GI)

    self_f, mean_f = _sc_gather_mean(nodes_r, neigh_r, feat_table)

    out = pl.pallas_call(
        _mm_body,
        grid=(B // _BM,),
        in_specs=[
            pl.BlockSpec((_BM, D), lambda i: (i, 0)),
            pl.BlockSpec((_BM, D), lambda i: (i, 0)),
            pl.BlockSpec((D, D), lambda i: (0, 0)),
            pl.BlockSpec((D, D), lambda i: (0, 0)),
        ],
        out_specs=pl.BlockSpec((_BM, D), lambda i: (i, 0)),
        out_shape=jax.ShapeDtypeStruct((B, D), jnp.float32),
    )(self_f, mean_f, weight[:D], weight[D:])
    return out
